# Initial kernel scaffold; baseline (speedup 1.0000x reference)
import functools

import jax
import jax.numpy as jnp
from jax import lax
from jax.experimental import pallas as pl
from jax.experimental.pallas import tpu as pltpu
from jax.experimental.pallas import tpu_sc as plsc

N = 10000
E = 320000
D = 128

NC = 2
NS = 16
NW = NC * NS
EW = E // NW
C = 125
NCH = EW // C
RS = N // NS
DW = 16

_mesh = plsc.VectorSubcoreMesh(core_axis_name="c", subcore_axis_name="s")


def _zero_vmem(buf, rows, width):
    per_row = width // 16

    def body(t, _):
        i = t // per_row
        j = (t % per_row) * 16
        buf[i, pl.ds(j, 16)] = jnp.zeros((16,), jnp.float32)
        return 0

    lax.fori_loop(0, rows * per_row, body, 0)


@functools.partial(
    pl.kernel,
    out_type=jax.ShapeDtypeStruct((NC, N, DW), jnp.float32),
    mesh=_mesh,
    scratch_types=[
        pltpu.VMEM((NCH, C), jnp.int32),
        pltpu.VMEM((C, DW), jnp.float32),
        pltpu.VMEM((RS, DW), jnp.float32),
        pltpu.VMEM_SHARED((N, DW), jnp.float32),
    ],
)
def _sc_deg(row_hbm, out_hbm, row_v, ones_v, buf_v, acc_sh):
    cid = lax.axis_index("c")
    sid = lax.axis_index("s")
    wid = sid * NC + cid

    pltpu.sync_copy(row_hbm.at[wid], row_v)

    def fill_ones(i, _):
        ones_v[i, :] = jnp.ones((DW,), jnp.float32)
        return 0

    lax.fori_loop(0, C, fill_ones, 0)

    _zero_vmem(buf_v, RS, DW)
    pltpu.sync_copy(buf_v, acc_sh.at[pl.ds(sid * RS, RS)])
    plsc.subcore_barrier()

    def chunk(k, _):
        pltpu.sync_copy(ones_v, acc_sh.at[row_v.at[k]], add=True)
        return 0

    lax.fori_loop(0, NCH, chunk, 0)
    plsc.subcore_barrier()

    pltpu.sync_copy(acc_sh.at[pl.ds(sid * RS, RS)], buf_v)
    pltpu.sync_copy(buf_v, out_hbm.at[cid, pl.ds(sid * RS, RS)])


@functools.partial(
    pl.kernel,
    out_type=jax.ShapeDtypeStruct((NC, N, D), jnp.float32),
    mesh=_mesh,
    scratch_types=[
        pltpu.VMEM((NCH, C), jnp.int32),
        pltpu.VMEM((NCH, C), jnp.int32),
        pltpu.VMEM((C, D), jnp.float32),
        pltpu.VMEM((C, D), jnp.float32),
        pltpu.VMEM((RS, D), jnp.float32),
        pltpu.VMEM_SHARED((N, D), jnp.float32),
        pltpu.SemaphoreType.DMA,
        pltpu.SemaphoreType.DMA,
    ],
)
def _sc_msg(h_hbm, row_hbm, col_hbm, out_hbm,
            row_v, col_v, rows_a, rows_b, buf_v, acc_sh, sem_a, sem_b):
    cid = lax.axis_index("c")
    sid = lax.axis_index("s")
    wid = sid * NC + cid

    pltpu.sync_copy(row_hbm.at[wid], row_v)
    pltpu.sync_copy(col_hbm.at[wid], col_v)

    _zero_vmem(buf_v, RS, D)
    pltpu.sync_copy(buf_v, acc_sh.at[pl.ds(sid * RS, RS)])
    plsc.subcore_barrier()

    pltpu.async_copy(h_hbm.at[row_v.at[0]], rows_a, sem_a).wait()

    def pair(kk, _):
        k = 2 * kk
        gb = pltpu.async_copy(h_hbm.at[row_v.at[k + 1]], rows_b, sem_b)
        pltpu.sync_copy(rows_a, acc_sh.at[col_v.at[k]], add=True)
        gb.wait()
        ga = pltpu.async_copy(h_hbm.at[row_v.at[(k + 2) % NCH]], rows_a, sem_a)
        pltpu.sync_copy(rows_b, acc_sh.at[col_v.at[k + 1]], add=True)
        ga.wait()
        return 0

    lax.fori_loop(0, NCH // 2, pair, 0)
    plsc.subcore_barrier()

    pltpu.sync_copy(acc_sh.at[pl.ds(sid * RS, RS)], buf_v)
    pltpu.sync_copy(buf_v, out_hbm.at[cid, pl.ds(sid * RS, RS)])


def _l0_body(x_ref, w_ref, b_ref, d0_ref, d1_ref, o_ref):
    dis = lax.rsqrt(d0_ref[...] + d1_ref[...])
    h = lax.dot_general(x_ref[...], w_ref[...], (((1,), (1,)), ((), ())),
                        precision=lax.Precision.HIGHEST)
    o_ref[...] = (h + b_ref[...]) * dis


def _l1_body(p0_ref, p1_ref, w_ref, b_ref, d0_ref, d1_ref, o_ref):
    dis = lax.rsqrt(d0_ref[...] + d1_ref[...])
    u = (p0_ref[...] + p1_ref[...]) * dis
    h = lax.dot_general(u, w_ref[...], (((1,), (1,)), ((), ())),
                        precision=lax.Precision.HIGHEST)
    o_ref[...] = (h + b_ref[...]) * dis


def _fin_body(p0_ref, p1_ref, d0_ref, d1_ref, o_ref):
    dis = lax.rsqrt(d0_ref[...] + d1_ref[...])
    o_ref[...] = (p0_ref[...] + p1_ref[...]) * dis


_out_nd = jax.ShapeDtypeStruct((N, D), jnp.float32)
_tc_l0 = pl.pallas_call(_l0_body, out_shape=_out_nd)
_tc_l1 = pl.pallas_call(_l1_body, out_shape=_out_nd)
_tc_fin = pl.pallas_call(_fin_body, out_shape=_out_nd)


def kernel(x, edge_index, W0, b0, W1, b1):
    row3 = edge_index[0].astype(jnp.int32).reshape(NW, NCH, C)
    col3 = edge_index[1].astype(jnp.int32).reshape(NW, NCH, C)
    b0r = b0.reshape(1, D)
    b1r = b1.reshape(1, D)

    degp = _sc_deg(row3)
    d0 = degp[0, :, 0:1]
    d1 = degp[1, :, 0:1]

    h0 = _tc_l0(x, W0, b0r, d0, d1)
    p0 = _sc_msg(h0, row3, col3)
    h1 = _tc_l1(p0[0], p0[1], W1, b1r, d0, d1)
    p1 = _sc_msg(h1, row3, col3)
    return _tc_fin(p1[0], p1[1], d0, d1)


# speedup vs baseline: 19.8690x; 19.8690x over previous
import functools

import jax
import jax.numpy as jnp
from jax import lax
from jax.experimental import pallas as pl
from jax.experimental.pallas import tpu as pltpu
from jax.experimental.pallas import tpu_sc as plsc

N = 10000
E = 320000
D = 128

NC = 2
NS = 16
NW = NC * NS
EW = E // NW
C = 80
NCH = EW // C
N_PAD = 10240
RS = N_PAD // NS
PR = 80
NP = RS // PR
DW = 16

_mesh = plsc.VectorSubcoreMesh(core_axis_name="c", subcore_axis_name="s")


def _zero_vmem(buf, rows, width):
    per_row = width // 16

    def body(t, _):
        i = t // per_row
        j = (t % per_row) * 16
        buf[i, pl.ds(j, 16)] = jnp.zeros((16,), jnp.float32)
        return 0

    lax.fori_loop(0, rows * per_row, body, 0)


@functools.partial(
    pl.kernel,
    out_type=jax.ShapeDtypeStruct((NC, N_PAD, DW), jnp.float32),
    mesh=_mesh,
    scratch_types=[
        pltpu.VMEM((NCH, C), jnp.int32),
        pltpu.VMEM((C, DW), jnp.float32),
        pltpu.VMEM((PR, DW), jnp.float32),
        pltpu.VMEM_SHARED((N_PAD, DW), jnp.float32),
    ],
)
def _sc_deg(row_hbm, out_hbm, row_v, ones_v, buf_v, acc_sh):
    cid = lax.axis_index("c")
    sid = lax.axis_index("s")
    wid = sid * NC + cid

    pltpu.sync_copy(row_hbm.at[wid], row_v)

    def fill_ones(i, _):
        ones_v[i, :] = jnp.ones((DW,), jnp.float32)
        return 0

    lax.fori_loop(0, C, fill_ones, 0)

    _zero_vmem(buf_v, PR, DW)

    def zpiece(t, _):
        off = pl.multiple_of(sid * RS + t * PR, PR)
        pltpu.sync_copy(buf_v, acc_sh.at[pl.ds(off, PR)])
        return 0

    lax.fori_loop(0, NP, zpiece, 0)
    plsc.subcore_barrier()

    def chunk(k, _):
        pltpu.sync_copy(ones_v, acc_sh.at[row_v.at[k]], add=True)
        return 0

    lax.fori_loop(0, NCH, chunk, 0)
    plsc.subcore_barrier()

    def wpiece(t, _):
        off = pl.multiple_of(sid * RS + t * PR, PR)
        pltpu.sync_copy(acc_sh.at[pl.ds(off, PR)], buf_v)
        pltpu.sync_copy(buf_v, out_hbm.at[cid, pl.ds(off, PR)])
        return 0

    lax.fori_loop(0, NP, wpiece, 0)


@functools.partial(
    pl.kernel,
    out_type=jax.ShapeDtypeStruct((NC, N_PAD, D), jnp.float32),
    mesh=_mesh,
    scratch_types=[
        pltpu.VMEM((EW,), jnp.int32),
        pltpu.VMEM((NCH, C), jnp.int32),
        pltpu.VMEM((C, D), jnp.float32),
        pltpu.VMEM((C, D), jnp.float32),
        pltpu.VMEM_SHARED((N_PAD, D), jnp.float32),
        pltpu.SemaphoreType.DMA,
        pltpu.SemaphoreType.DMA,
    ],
)
def _sc_msg(h_hbm, row_hbm, col_hbm, out_hbm,
            row_v, col_v, rows_a, rows_b, acc_sh, sem_a, sem_b):
    cid = lax.axis_index("c")
    sid = lax.axis_index("s")
    wid = sid * NC + cid

    pltpu.sync_copy(row_hbm.at[wid], row_v)
    pltpu.sync_copy(col_hbm.at[wid], col_v)

    _zero_vmem(rows_a, PR, D)

    def zpiece(t, _):
        off = pl.multiple_of(sid * RS + t * PR, PR)
        pltpu.sync_copy(rows_a.at[pl.ds(0, PR)], acc_sh.at[pl.ds(off, PR)])
        return 0

    lax.fori_loop(0, NP, zpiece, 0)
    plsc.subcore_barrier()

    def ridx(k):
        return row_v.at[pl.ds(pl.multiple_of(k * C, 8), C)]

    pltpu.async_copy(h_hbm.at[ridx(0)], rows_a, sem_a).wait()

    def pair(kk, _):
        k = 2 * kk
        gb = pltpu.async_copy(h_hbm.at[ridx(k + 1)], rows_b, sem_b)
        pltpu.sync_copy(rows_a, acc_sh.at[col_v.at[k]], add=True)
        gb.wait()
        ga = pltpu.async_copy(h_hbm.at[ridx(k + 2)], rows_a, sem_a)
        pltpu.sync_copy(rows_b, acc_sh.at[col_v.at[k + 1]], add=True)
        ga.wait()
        return 0

    lax.fori_loop(0, NCH // 2, pair, 0)
    pltpu.sync_copy(rows_a, acc_sh.at[col_v.at[NCH - 1]], add=True)
    plsc.subcore_barrier()

    def wpiece(t, _):
        off = pl.multiple_of(sid * RS + t * PR, PR)
        pltpu.sync_copy(acc_sh.at[pl.ds(off, PR)], rows_a.at[pl.ds(0, PR)])
        pltpu.sync_copy(rows_a.at[pl.ds(0, PR)], out_hbm.at[cid, pl.ds(off, PR)])
        return 0

    lax.fori_loop(0, NP, wpiece, 0)


def _l0_body(x_ref, w_ref, b_ref, d0_ref, d1_ref, o_ref):
    dis = lax.rsqrt(d0_ref[...] + d1_ref[...])
    h = lax.dot_general(x_ref[...], w_ref[...], (((1,), (1,)), ((), ())),
                        precision=lax.Precision.HIGHEST)
    o_ref[...] = (h + b_ref[...]) * dis


def _l1_body(p0_ref, p1_ref, w_ref, b_ref, d0_ref, d1_ref, o_ref):
    dis = lax.rsqrt(d0_ref[...] + d1_ref[...])
    u = (p0_ref[...] + p1_ref[...]) * dis
    h = lax.dot_general(u, w_ref[...], (((1,), (1,)), ((), ())),
                        precision=lax.Precision.HIGHEST)
    o_ref[...] = (h + b_ref[...]) * dis


def _fin_body(p0_ref, p1_ref, d0_ref, d1_ref, o_ref):
    dis = lax.rsqrt(d0_ref[...] + d1_ref[...])
    o_ref[...] = (p0_ref[...] + p1_ref[...]) * dis


_out_nd = jax.ShapeDtypeStruct((N, D), jnp.float32)
_tc_l0 = pl.pallas_call(_l0_body, out_shape=_out_nd)
_tc_l1 = pl.pallas_call(_l1_body, out_shape=_out_nd)
_tc_fin = pl.pallas_call(_fin_body, out_shape=_out_nd)


def kernel(x, edge_index, W0, b0, W1, b1):
    row2 = edge_index[0].astype(jnp.int32).reshape(NW, EW)
    row3 = edge_index[0].astype(jnp.int32).reshape(NW, NCH, C)
    col3 = edge_index[1].astype(jnp.int32).reshape(NW, NCH, C)
    b0r = b0.reshape(1, D)
    b1r = b1.reshape(1, D)

    degp = _sc_deg(row3)
    d0 = degp[0, :N, 0:1]
    d1 = degp[1, :N, 0:1]

    h0 = _tc_l0(x, W0, b0r, d0, d1)
    p0 = _sc_msg(h0, row2, col3)
    h1 = _tc_l1(p0[0, :N], p0[1, :N], W1, b1r, d0, d1)
    p1 = _sc_msg(h1, row2, col3)
    return _tc_fin(p1[0, :N], p1[1, :N], d0, d1)
